# initial kernel scaffold (unmeasured)
import jax
import jax.numpy as jnp
from jax import lax
from jax.experimental import pallas as pl
from jax.experimental.pallas import tpu as pltpu

N_DEV = 4
SQ = 2048
D = 1024
HQ = 32
HP = 8
DH = 128
NG = 4
GQ = SQ // NG
GK = 2048 // NG
QB = SQ // 64 // NG
SCALE = 0.08838834764831843


def _body(xg_ref, wq_ref, kh_ref, vh_ref, wo_ref, out_ref,
          w_all, send_sems, recv_sems):
    my = lax.axis_index("i")
    left = lax.rem(my + (N_DEV - 1), N_DEV)
    right = lax.rem(my + 1, N_DEV)

    barrier = pltpu.get_barrier_semaphore()
    pl.semaphore_signal(barrier, inc=1, device_id=(left,),
                        device_id_type=pl.DeviceIdType.MESH)
    pl.semaphore_signal(barrier, inc=1, device_id=(right,),
                        device_id_type=pl.DeviceIdType.MESH)
    pl.semaphore_wait(barrier, 2)

    pl.store(w_all, (pl.ds(my, 1), pl.ds(0, 1), slice(None), slice(None)),
             wq_ref[...].reshape(1, 1, D, D))
    pl.store(w_all, (pl.ds(my, 1), pl.ds(1, 1), slice(None), slice(None)),
             wo_ref[...].reshape(1, 1, D, D))

    sends = []
    for s in range(N_DEV):
        j = lax.rem(my + (N_DEV - s), N_DEV)
        if s > 0:
            h = s - 1
            recv = pltpu.make_async_remote_copy(
                src_ref=w_all.at[pl.ds(j, 1)],
                dst_ref=w_all.at[pl.ds(j, 1)],
                send_sem=send_sems.at[h],
                recv_sem=recv_sems.at[h],
                device_id=(left,),
                device_id_type=pl.DeviceIdType.MESH,
            )
            recv.wait_recv()
        if s < N_DEV - 1:
            send = pltpu.make_async_remote_copy(
                src_ref=w_all.at[pl.ds(j, 1)],
                dst_ref=w_all.at[pl.ds(j, 1)],
                send_sem=send_sems.at[s],
                recv_sem=recv_sems.at[s],
                device_id=(right,),
                device_id_type=pl.DeviceIdType.MESH,
            )
            send.start()
            sends.append(send)

        wq_j = pl.load(
            w_all, (pl.ds(j, 1), pl.ds(0, 1), slice(None), slice(None))
        ).reshape(D, D)
        wo_j = pl.load(
            w_all, (pl.ds(j, 1), pl.ds(1, 1), slice(None), slice(None))
        ).reshape(D, D)
        for r in range(NG):
            xr = xg_ref[r]
            q = (lax.dot_general(xr, wq_j, (((1,), (0,)), ((), ())),
                                 preferred_element_type=jnp.float32)
                 * SCALE).astype(jnp.bfloat16)
            ctx_parts = []
            for hh in range(HP):
                qh = q[:, hh * DH:(hh + 1) * DH]
                khr = pl.load(
                    kh_ref,
                    (pl.ds(j * HP + hh, 1), pl.ds(r, 1),
                     slice(None), slice(None)),
                ).reshape(GK, DH)
                vhr = pl.load(
                    vh_ref,
                    (pl.ds(j * HP + hh, 1), pl.ds(r, 1),
                     slice(None), slice(None)),
                ).reshape(GK, DH)
                sc = lax.dot_general(qh, khr, (((1,), (1,)), ((), ())),
                                     preferred_element_type=jnp.float32)
                m = jnp.max(sc, axis=-1, keepdims=True)
                p = jnp.exp(sc - m)
                p = (p / jnp.sum(p, axis=-1, keepdims=True)
                     ).astype(jnp.bfloat16)
                ctx_parts.append(
                    lax.dot_general(p, vhr, (((1,), (0,)), ((), ())),
                                    preferred_element_type=jnp.float32
                                    ).astype(jnp.bfloat16))
            ctx = jnp.concatenate(ctx_parts, axis=1)
            acc = lax.dot_general(ctx, wo_j, (((1,), (0,)), ((), ())),
                                  preferred_element_type=jnp.float32)
            if s == 0:
                out_ref[r, :, :] = acc
            else:
                out_ref[r, :, :] = out_ref[r, :, :] + acc

    for send in sends:
        send.wait_send()


def kernel(x, Wq, K_ext, V_ext, Wo):
    xg = (x[0].astype(jnp.bfloat16)
          .reshape(QB, NG, 64, D)
          .transpose(1, 0, 2, 3)
          .reshape(NG, GQ, D))
    wq = Wq.astype(jnp.bfloat16)
    wo = Wo.astype(jnp.bfloat16)
    kh = (K_ext[0].astype(jnp.bfloat16)
          .reshape(8, NG, 64, HQ, DH)
          .transpose(3, 1, 0, 2, 4)
          .reshape(HQ, NG, GK, DH))
    vh = (V_ext[0].astype(jnp.bfloat16)
          .reshape(8, NG, 64, HQ, DH)
          .transpose(3, 1, 0, 2, 4)
          .reshape(HQ, NG, GK, DH))

    outg = pl.pallas_call(
        _body,
        out_shape=jax.ShapeDtypeStruct((NG, GQ, D), jnp.float32),
        in_specs=[pl.BlockSpec(memory_space=pltpu.VMEM)] * 5,
        out_specs=pl.BlockSpec(memory_space=pltpu.VMEM),
        scratch_shapes=[
            pltpu.VMEM((N_DEV, 2, D, D), jnp.bfloat16),
            pltpu.SemaphoreType.DMA((N_DEV - 1,)),
            pltpu.SemaphoreType.DMA((N_DEV - 1,)),
        ],
        compiler_params=pltpu.CompilerParams(collective_id=0),
    )(xg, wq, kh, vh, wo)

    out = (outg.reshape(NG, QB, 64, D)
           .transpose(1, 0, 2, 3)
           .reshape(1, SQ, D))
    return out


# baseline (device time: 250257 ns/iter reference)
import jax
import jax.numpy as jnp
from jax import lax
from jax.experimental import pallas as pl
from jax.experimental.pallas import tpu as pltpu

N_DEV = 4
SQ = 2048
D = 1024
HQ = 32
HP = 8
DH = 128
NG = 4
GQ = SQ // NG
GK = 2048 // NG
QB = SQ // 64 // NG
SCALE = 0.08838834764831843


def _body(xg_ref, wq_ref, kh_ref, vh_ref, wo_ref, out_ref,
          w_all, k_buf, v_buf, send_sems, recv_sems, k_sems, v_sems):
    my = lax.axis_index("i")
    left = lax.rem(my + (N_DEV - 1), N_DEV)
    right = lax.rem(my + 1, N_DEV)

    def dma_k(slot, j):
        return pltpu.make_async_copy(
            kh_ref.at[pl.ds(j * HP, HP)], k_buf.at[slot], k_sems.at[slot])

    def dma_v(slot, j):
        return pltpu.make_async_copy(
            vh_ref.at[pl.ds(j * HP, HP)], v_buf.at[slot], v_sems.at[slot])

    dma_k(0, my).start()
    dma_v(0, my).start()

    barrier = pltpu.get_barrier_semaphore()
    pl.semaphore_signal(barrier, inc=1, device_id=(left,),
                        device_id_type=pl.DeviceIdType.MESH)
    pl.semaphore_signal(barrier, inc=1, device_id=(right,),
                        device_id_type=pl.DeviceIdType.MESH)
    pl.semaphore_wait(barrier, 2)

    w_all[my, 0] = wq_ref[...]
    w_all[my, 1] = wo_ref[...]

    sends = []
    for s in range(N_DEV):
        j = lax.rem(my + (N_DEV - s), N_DEV)
        slot = s % 2
        if s > 0:
            h = s - 1
            recv = pltpu.make_async_remote_copy(
                src_ref=w_all.at[pl.ds(j, 1)],
                dst_ref=w_all.at[pl.ds(j, 1)],
                send_sem=send_sems.at[h],
                recv_sem=recv_sems.at[h],
                device_id=(left,),
                device_id_type=pl.DeviceIdType.MESH,
            )
            recv.wait_recv()
        if s < N_DEV - 1:
            send = pltpu.make_async_remote_copy(
                src_ref=w_all.at[pl.ds(j, 1)],
                dst_ref=w_all.at[pl.ds(j, 1)],
                send_sem=send_sems.at[s],
                recv_sem=recv_sems.at[s],
                device_id=(right,),
                device_id_type=pl.DeviceIdType.MESH,
            )
            send.start()
            sends.append(send)
            jn = lax.rem(my + (N_DEV - s - 1), N_DEV)
            dma_k(1 - slot, jn).start()
            dma_v(1 - slot, jn).start()

        dma_k(slot, j).wait()
        dma_v(slot, j).wait()

        wq_j = w_all[j, 0]
        wo_j = w_all[j, 1]
        for r in range(NG):
            xr = xg_ref[r]
            q = (lax.dot_general(xr, wq_j, (((1,), (0,)), ((), ())),
                                 preferred_element_type=jnp.float32)
                 * SCALE).astype(jnp.bfloat16)
            ctx_parts = []
            for hh in range(HP):
                qh = q[:, hh * DH:(hh + 1) * DH]
                khr = k_buf[slot, hh, r]
                vhr = v_buf[slot, hh, r]
                sc = lax.dot_general(qh, khr, (((1,), (1,)), ((), ())),
                                     preferred_element_type=jnp.float32)
                m = jnp.max(sc, axis=-1, keepdims=True)
                p = jnp.exp(sc - m)
                p = (p / jnp.sum(p, axis=-1, keepdims=True)
                     ).astype(jnp.bfloat16)
                ctx_parts.append(
                    lax.dot_general(p, vhr, (((1,), (0,)), ((), ())),
                                    preferred_element_type=jnp.float32
                                    ).astype(jnp.bfloat16))
            ctx = jnp.concatenate(ctx_parts, axis=1)
            acc = lax.dot_general(ctx, wo_j, (((1,), (0,)), ((), ())),
                                  preferred_element_type=jnp.float32)
            if s == 0:
                out_ref[r, :, :] = acc
            else:
                out_ref[r, :, :] = out_ref[r, :, :] + acc

    for send in sends:
        send.wait_send()


def kernel(x, Wq, K_ext, V_ext, Wo):
    xg = (x[0].astype(jnp.bfloat16)
          .reshape(QB, NG, 64, D)
          .transpose(1, 0, 2, 3)
          .reshape(NG, GQ, D))
    wq = Wq.astype(jnp.bfloat16)
    wo = Wo.astype(jnp.bfloat16)
    kh = (K_ext[0].astype(jnp.bfloat16)
          .reshape(8, NG, 64, HQ, DH)
          .transpose(3, 1, 0, 2, 4)
          .reshape(HQ, NG, GK, DH))
    vh = (V_ext[0].astype(jnp.bfloat16)
          .reshape(8, NG, 64, HQ, DH)
          .transpose(3, 1, 0, 2, 4)
          .reshape(HQ, NG, GK, DH))

    outg = pl.pallas_call(
        _body,
        out_shape=jax.ShapeDtypeStruct((NG, GQ, D), jnp.float32),
        in_specs=[
            pl.BlockSpec(memory_space=pltpu.VMEM),
            pl.BlockSpec(memory_space=pltpu.VMEM),
            pl.BlockSpec(memory_space=pl.ANY),
            pl.BlockSpec(memory_space=pl.ANY),
            pl.BlockSpec(memory_space=pltpu.VMEM),
        ],
        out_specs=pl.BlockSpec(memory_space=pltpu.VMEM),
        scratch_shapes=[
            pltpu.VMEM((N_DEV, 2, D, D), jnp.bfloat16),
            pltpu.VMEM((2, HP, NG, GK, DH), jnp.bfloat16),
            pltpu.VMEM((2, HP, NG, GK, DH), jnp.bfloat16),
            pltpu.SemaphoreType.DMA((N_DEV - 1,)),
            pltpu.SemaphoreType.DMA((N_DEV - 1,)),
            pltpu.SemaphoreType.DMA((2,)),
            pltpu.SemaphoreType.DMA((2,)),
        ],
        compiler_params=pltpu.CompilerParams(
            collective_id=0,
            vmem_limit_bytes=100 * 1024 * 1024,
        ),
    )(xg, wq, kh, vh, wo)

    out = (outg.reshape(NG, QB, 64, D)
           .transpose(1, 0, 2, 3)
           .reshape(1, SQ, D))
    return out


# device time: 203883 ns/iter; 1.2275x vs baseline; 1.2275x over previous
import jax
import jax.numpy as jnp
from jax import lax
from jax.experimental import pallas as pl
from jax.experimental.pallas import tpu as pltpu

N_DEV = 4
SQ = 2048
D = 1024
HQ = 32
HP = 8
DH = 128
NG = 4
GQ = SQ // NG
GK = 2048 // NG
QB = SQ // 64 // NG
SCALE = 0.08838834764831843


def _body(xg_ref, wq_ref, kh_ref, vh_ref, wo_ref, out_ref,
          w_all, k_buf, v_buf, send_sems, recv_sems, k_sems, v_sems):
    my = lax.axis_index("i")
    left = lax.rem(my + (N_DEV - 1), N_DEV)
    right = lax.rem(my + 1, N_DEV)

    def dma_k(slot, j):
        return pltpu.make_async_copy(
            kh_ref.at[pl.ds(j * HP, HP)], k_buf.at[slot], k_sems.at[slot])

    def dma_v(slot, j):
        return pltpu.make_async_copy(
            vh_ref.at[pl.ds(j * HP, HP)], v_buf.at[slot], v_sems.at[slot])

    dma_k(0, my).start()
    dma_v(0, my).start()
    dma_k(1, left).start()
    dma_v(1, left).start()

    ldiag = lax.rem(my + 2, N_DEV)

    barrier = pltpu.get_barrier_semaphore()
    for nbr in (left, right, ldiag):
        pl.semaphore_signal(barrier, inc=1, device_id=(nbr,),
                            device_id_type=pl.DeviceIdType.MESH)
    pl.semaphore_wait(barrier, 3)

    w_all[my, 0, 0] = wq_ref[: D // 2]
    w_all[my, 1, 0] = wq_ref[D // 2:]
    w_all[my, 0, 1] = wo_ref[: D // 2]
    w_all[my, 1, 1] = wo_ref[D // 2:]

    def rdma(src, dst, ssem, rsem, target):
        return pltpu.make_async_remote_copy(
            src_ref=src, dst_ref=dst,
            send_sem=send_sems.at[ssem], recv_sem=recv_sems.at[rsem],
            device_id=(target,), device_id_type=pl.DeviceIdType.MESH,
        )

    mine = w_all.at[pl.ds(my, 1)]
    send_to_left = rdma(mine, mine, 0, 1, left)
    send_to_right = rdma(mine, mine, 1, 0, right)
    send_to_diag = rdma(mine, mine, 2, 2, ldiag)
    send_to_left.start()
    send_to_right.start()
    send_to_diag.start()

    def compute_chunk(j, slot, first):
        dma_k(slot, j).wait()
        dma_v(slot, j).wait()
        wq_j = jnp.concatenate([w_all[j, 0, 0], w_all[j, 1, 0]], axis=0)
        wo_j = jnp.concatenate([w_all[j, 0, 1], w_all[j, 1, 1]], axis=0)
        for r in range(NG):
            xr = xg_ref[r]
            q = (lax.dot_general(xr, wq_j, (((1,), (0,)), ((), ())),
                                 preferred_element_type=jnp.float32)
                 * SCALE).astype(jnp.bfloat16)
            ctx_parts = []
            for hh in range(HP):
                qh = q[:, hh * DH:(hh + 1) * DH]
                khr = k_buf[slot, hh, r]
                vhr = v_buf[slot, hh, r]
                sc = lax.dot_general(qh, khr, (((1,), (1,)), ((), ())),
                                     preferred_element_type=jnp.float32)
                m = jnp.max(sc, axis=-1, keepdims=True)
                p = jnp.exp(sc - m)
                p = (p / jnp.sum(p, axis=-1, keepdims=True)
                     ).astype(jnp.bfloat16)
                ctx_parts.append(
                    lax.dot_general(p, vhr, (((1,), (0,)), ((), ())),
                                    preferred_element_type=jnp.float32
                                    ).astype(jnp.bfloat16))
            ctx = jnp.concatenate(ctx_parts, axis=1)
            acc = lax.dot_general(ctx, wo_j, (((1,), (0,)), ((), ())),
                                  preferred_element_type=jnp.float32)
            if first:
                out_ref[r, :, :] = acc
            else:
                out_ref[r, :, :] = out_ref[r, :, :] + acc

    compute_chunk(my, 0, first=True)
    dma_k(0, right).start()
    dma_v(0, right).start()

    lref = w_all.at[pl.ds(left, 1)]
    rdma(lref, lref, 0, 0, left).wait_recv()
    compute_chunk(left, 1, first=False)
    dma_k(1, ldiag).start()
    dma_v(1, ldiag).start()

    rref = w_all.at[pl.ds(right, 1)]
    rdma(rref, rref, 1, 1, right).wait_recv()
    compute_chunk(right, 0, first=False)

    dref = w_all.at[pl.ds(ldiag, 1)]
    rdma(dref, dref, 2, 2, ldiag).wait_recv()
    compute_chunk(ldiag, 1, first=False)

    send_to_left.wait_send()
    send_to_right.wait_send()
    send_to_diag.wait_send()


def kernel(x, Wq, K_ext, V_ext, Wo):
    xg = (x[0].astype(jnp.bfloat16)
          .reshape(QB, NG, 64, D)
          .transpose(1, 0, 2, 3)
          .reshape(NG, GQ, D))
    wq = Wq.astype(jnp.bfloat16)
    wo = Wo.astype(jnp.bfloat16)
    kh = (K_ext[0].astype(jnp.bfloat16)
          .reshape(8, NG, 64, HQ, DH)
          .transpose(3, 1, 0, 2, 4)
          .reshape(HQ, NG, GK, DH))
    vh = (V_ext[0].astype(jnp.bfloat16)
          .reshape(8, NG, 64, HQ, DH)
          .transpose(3, 1, 0, 2, 4)
          .reshape(HQ, NG, GK, DH))

    outg = pl.pallas_call(
        _body,
        out_shape=jax.ShapeDtypeStruct((NG, GQ, D), jnp.float32),
        in_specs=[
            pl.BlockSpec(memory_space=pltpu.VMEM),
            pl.BlockSpec(memory_space=pltpu.VMEM),
            pl.BlockSpec(memory_space=pl.ANY),
            pl.BlockSpec(memory_space=pl.ANY),
            pl.BlockSpec(memory_space=pltpu.VMEM),
        ],
        out_specs=pl.BlockSpec(memory_space=pltpu.VMEM),
        scratch_shapes=[
            pltpu.VMEM((N_DEV, 2, 2, D // 2, D), jnp.bfloat16),
            pltpu.VMEM((2, HP, NG, GK, DH), jnp.bfloat16),
            pltpu.VMEM((2, HP, NG, GK, DH), jnp.bfloat16),
            pltpu.SemaphoreType.DMA((4,)),
            pltpu.SemaphoreType.DMA((4,)),
            pltpu.SemaphoreType.DMA((2,)),
            pltpu.SemaphoreType.DMA((2,)),
        ],
        compiler_params=pltpu.CompilerParams(
            collective_id=0,
            vmem_limit_bytes=100 * 1024 * 1024,
        ),
    )(xg, wq, kh, vh, wo)

    out = (outg.reshape(NG, QB, 64, D)
           .transpose(1, 0, 2, 3)
           .reshape(1, SQ, D))
    return out


# device time: 202827 ns/iter; 1.2338x vs baseline; 1.0052x over previous
import jax
import jax.numpy as jnp
from jax import lax
from jax.experimental import pallas as pl
from jax.experimental.pallas import tpu as pltpu

N_DEV = 4
SQ = 2048
D = 1024
HQ = 32
HP = 8
DH = 128
NG = 4
GQ = SQ // NG
GK = 2048 // NG
QB = SQ // 64 // NG
SCALE = 0.08838834764831843


def _body(xg_ref, wq_ref, kh_ref, vh_ref, wo_ref, out_ref,
          w_all, k_buf, v_buf, send_sems, recv_sems, k_sems, v_sems):
    my = lax.axis_index("i")
    left = lax.rem(my + (N_DEV - 1), N_DEV)
    right = lax.rem(my + 1, N_DEV)

    def dma_k(slot, j):
        return pltpu.make_async_copy(
            kh_ref.at[pl.ds(j * HP, HP)], k_buf.at[slot], k_sems.at[slot])

    def dma_v(slot, j):
        return pltpu.make_async_copy(
            vh_ref.at[pl.ds(j * HP, HP)], v_buf.at[slot], v_sems.at[slot])

    dma_k(0, my).start()
    dma_v(0, my).start()
    dma_k(1, left).start()
    dma_v(1, left).start()

    ldiag = lax.rem(my + 2, N_DEV)

    barrier = pltpu.get_barrier_semaphore()
    for nbr in (left, right):
        pl.semaphore_signal(barrier, inc=1, device_id=(nbr,),
                            device_id_type=pl.DeviceIdType.MESH)
    pl.semaphore_wait(barrier, 2)

    w_all[my, 0, 0] = wq_ref[: D // 2]
    w_all[my, 1, 0] = wq_ref[D // 2:]
    w_all[my, 0, 1] = wo_ref[: D // 2]
    w_all[my, 1, 1] = wo_ref[D // 2:]

    def rdma(src, dst, ssem, rsem, target):
        return pltpu.make_async_remote_copy(
            src_ref=src, dst_ref=dst,
            send_sem=send_sems.at[ssem], recv_sem=recv_sems.at[rsem],
            device_id=(target,), device_id_type=pl.DeviceIdType.MESH,
        )

    mine = w_all.at[pl.ds(my, 1)]
    send_to_left = rdma(mine, mine, 0, 1, left)
    send_to_right = rdma(mine, mine, 1, 0, right)
    send_to_left.start()
    send_to_right.start()

    def half(j, h):
        return w_all.at[pl.ds(j, 1), pl.ds(h, 1)]

    def compute_chunk(j, slot, first):
        dma_k(slot, j).wait()
        dma_v(slot, j).wait()
        wq_j = jnp.concatenate([w_all[j, 0, 0], w_all[j, 1, 0]], axis=0)
        wo_j = jnp.concatenate([w_all[j, 0, 1], w_all[j, 1, 1]], axis=0)
        for r in range(NG):
            xr = xg_ref[r]
            q = (lax.dot_general(xr, wq_j, (((1,), (0,)), ((), ())),
                                 preferred_element_type=jnp.float32)
                 * SCALE).astype(jnp.bfloat16)
            ctx_parts = []
            for hh in range(HP):
                qh = q[:, hh * DH:(hh + 1) * DH]
                khr = k_buf[slot, hh, r]
                vhr = v_buf[slot, hh, r]
                sc = lax.dot_general(qh, khr, (((1,), (1,)), ((), ())),
                                     preferred_element_type=jnp.float32)
                m = jnp.max(sc, axis=-1, keepdims=True)
                p = jnp.exp(sc - m)
                p = (p / jnp.sum(p, axis=-1, keepdims=True)
                     ).astype(jnp.bfloat16)
                ctx_parts.append(
                    lax.dot_general(p, vhr, (((1,), (0,)), ((), ())),
                                    preferred_element_type=jnp.float32
                                    ).astype(jnp.bfloat16))
            ctx = jnp.concatenate(ctx_parts, axis=1)
            acc = lax.dot_general(ctx, wo_j, (((1,), (0,)), ((), ())),
                                  preferred_element_type=jnp.float32)
            if first:
                out_ref[r, :, :] = acc
            else:
                out_ref[r, :, :] = out_ref[r, :, :] + acc

    compute_chunk(my, 0, first=True)
    dma_k(0, right).start()
    dma_v(0, right).start()

    lref = w_all.at[pl.ds(left, 1)]
    rdma(lref, lref, 0, 0, left).wait_recv()
    fwd_right = rdma(half(left, 0), half(left, 0), 2, 2, right)
    fwd_right.start()
    compute_chunk(left, 1, first=False)
    dma_k(1, ldiag).start()
    dma_v(1, ldiag).start()

    rref = w_all.at[pl.ds(right, 1)]
    rdma(rref, rref, 1, 1, right).wait_recv()
    fwd_left = rdma(half(right, 1), half(right, 1), 3, 3, left)
    fwd_left.start()
    compute_chunk(right, 0, first=False)

    rdma(half(ldiag, 0), half(ldiag, 0), 2, 2, left).wait_recv()
    rdma(half(ldiag, 1), half(ldiag, 1), 3, 3, right).wait_recv()
    compute_chunk(ldiag, 1, first=False)

    send_to_left.wait_send()
    send_to_right.wait_send()
    fwd_right.wait_send()
    fwd_left.wait_send()


def kernel(x, Wq, K_ext, V_ext, Wo):
    xg = (x[0].astype(jnp.bfloat16)
          .reshape(QB, NG, 64, D)
          .transpose(1, 0, 2, 3)
          .reshape(NG, GQ, D))
    wq = Wq.astype(jnp.bfloat16)
    wo = Wo.astype(jnp.bfloat16)
    kh = (K_ext[0].astype(jnp.bfloat16)
          .reshape(8, NG, 64, HQ, DH)
          .transpose(3, 1, 0, 2, 4)
          .reshape(HQ, NG, GK, DH))
    vh = (V_ext[0].astype(jnp.bfloat16)
          .reshape(8, NG, 64, HQ, DH)
          .transpose(3, 1, 0, 2, 4)
          .reshape(HQ, NG, GK, DH))

    outg = pl.pallas_call(
        _body,
        out_shape=jax.ShapeDtypeStruct((NG, GQ, D), jnp.float32),
        in_specs=[
            pl.BlockSpec(memory_space=pltpu.VMEM),
            pl.BlockSpec(memory_space=pltpu.VMEM),
            pl.BlockSpec(memory_space=pl.ANY),
            pl.BlockSpec(memory_space=pl.ANY),
            pl.BlockSpec(memory_space=pltpu.VMEM),
        ],
        out_specs=pl.BlockSpec(memory_space=pltpu.VMEM),
        scratch_shapes=[
            pltpu.VMEM((N_DEV, 2, 2, D // 2, D), jnp.bfloat16),
            pltpu.VMEM((2, HP, NG, GK, DH), jnp.bfloat16),
            pltpu.VMEM((2, HP, NG, GK, DH), jnp.bfloat16),
            pltpu.SemaphoreType.DMA((4,)),
            pltpu.SemaphoreType.DMA((4,)),
            pltpu.SemaphoreType.DMA((2,)),
            pltpu.SemaphoreType.DMA((2,)),
        ],
        compiler_params=pltpu.CompilerParams(
            collective_id=0,
            vmem_limit_bytes=100 * 1024 * 1024,
        ),
    )(xg, wq, kh, vh, wo)

    out = (outg.reshape(NG, QB, 64, D)
           .transpose(1, 0, 2, 3)
           .reshape(1, SQ, D))
    return out


# device time: 197881 ns/iter; 1.2647x vs baseline; 1.0250x over previous
import jax
import jax.numpy as jnp
from jax import lax
from jax.experimental import pallas as pl
from jax.experimental.pallas import tpu as pltpu

N_DEV = 4
SQ = 2048
D = 1024
HQ = 32
HP = 8
DH = 128
NG = 4
GQ = SQ // NG
GK = 2048 // NG
QB = SQ // 64 // NG
SCALE = 0.08838834764831843


def _body(xg_ref, wq_ref, kh_ref, vh_ref, wo_ref, out_ref,
          w_all, k_buf, v_buf, send_sems, recv_sems, k_sems, v_sems):
    my = lax.axis_index("i")
    left = lax.rem(my + (N_DEV - 1), N_DEV)
    right = lax.rem(my + 1, N_DEV)

    def dma_k(slot, j):
        return pltpu.make_async_copy(
            kh_ref.at[pl.ds(j * HP, HP)], k_buf.at[slot], k_sems.at[slot])

    def dma_v(slot, j):
        return pltpu.make_async_copy(
            vh_ref.at[pl.ds(j * HP, HP)], v_buf.at[slot], v_sems.at[slot])

    dma_k(0, my).start()
    dma_v(0, my).start()
    dma_k(1, left).start()
    dma_v(1, left).start()

    ldiag = lax.rem(my + 2, N_DEV)

    barrier = pltpu.get_barrier_semaphore()
    for nbr in (left, right):
        pl.semaphore_signal(barrier, inc=1, device_id=(nbr,),
                            device_id_type=pl.DeviceIdType.MESH)
    pl.semaphore_wait(barrier, 2)

    w_all[my, 0, 0] = wq_ref[: D // 2]
    w_all[my, 1, 0] = wq_ref[D // 2:]
    w_all[my, 0, 1] = wo_ref[: D // 2]
    w_all[my, 1, 1] = wo_ref[D // 2:]

    def rdma(src, dst, ssem, rsem, target):
        return pltpu.make_async_remote_copy(
            src_ref=src, dst_ref=dst,
            send_sem=send_sems.at[ssem], recv_sem=recv_sems.at[rsem],
            device_id=(target,), device_id_type=pl.DeviceIdType.MESH,
        )

    mine = w_all.at[pl.ds(my, 1)]
    send_to_left = rdma(mine, mine, 0, 1, left)
    send_to_right = rdma(mine, mine, 1, 0, right)
    send_to_left.start()
    send_to_right.start()

    def half(j, h):
        return w_all.at[pl.ds(j, 1), pl.ds(h, 1)]

    def compute_chunk(j, slot, first):
        dma_k(slot, j).wait()
        dma_v(slot, j).wait()
        wq_j = jnp.concatenate([w_all[j, 0, 0], w_all[j, 1, 0]], axis=0)
        wo_j = jnp.concatenate([w_all[j, 0, 1], w_all[j, 1, 1]], axis=0)
        for r in range(NG):
            xr = xg_ref[r]
            q = lax.dot_general(xr, wq_j, (((1,), (0,)), ((), ())),
                                preferred_element_type=jnp.float32
                                ).astype(jnp.bfloat16)
            ctx_parts = []
            for hh in range(HP):
                qh = q[:, hh * DH:(hh + 1) * DH]
                khr = k_buf[slot, hh, r]
                vhr = v_buf[slot, hh, r]
                sc = lax.dot_general(qh, khr, (((1,), (1,)), ((), ())),
                                     preferred_element_type=jnp.float32)
                p = jnp.exp(sc)
                denom = jnp.sum(p, axis=-1, keepdims=True)
                ctx_parts.append(
                    (lax.dot_general(p.astype(jnp.bfloat16), vhr,
                                     (((1,), (0,)), ((), ())),
                                     preferred_element_type=jnp.float32)
                     / denom).astype(jnp.bfloat16))
            ctx = jnp.concatenate(ctx_parts, axis=1)
            acc = lax.dot_general(ctx, wo_j, (((1,), (0,)), ((), ())),
                                  preferred_element_type=jnp.float32)
            if first:
                out_ref[r, :, :] = acc
            else:
                out_ref[r, :, :] = out_ref[r, :, :] + acc

    compute_chunk(my, 0, first=True)
    dma_k(0, right).start()
    dma_v(0, right).start()

    lref = w_all.at[pl.ds(left, 1)]
    rdma(lref, lref, 0, 0, left).wait_recv()
    fwd_right = rdma(half(left, 0), half(left, 0), 2, 2, right)
    fwd_right.start()
    compute_chunk(left, 1, first=False)
    dma_k(1, ldiag).start()
    dma_v(1, ldiag).start()

    rref = w_all.at[pl.ds(right, 1)]
    rdma(rref, rref, 1, 1, right).wait_recv()
    fwd_left = rdma(half(right, 1), half(right, 1), 3, 3, left)
    fwd_left.start()
    compute_chunk(right, 0, first=False)

    rdma(half(ldiag, 0), half(ldiag, 0), 2, 2, left).wait_recv()
    rdma(half(ldiag, 1), half(ldiag, 1), 3, 3, right).wait_recv()
    compute_chunk(ldiag, 1, first=False)

    send_to_left.wait_send()
    send_to_right.wait_send()
    fwd_right.wait_send()
    fwd_left.wait_send()


def kernel(x, Wq, K_ext, V_ext, Wo):
    xg = (x[0].astype(jnp.bfloat16)
          .reshape(QB, NG, 64, D)
          .transpose(1, 0, 2, 3)
          .reshape(NG, GQ, D))
    wq = (Wq * SCALE).astype(jnp.bfloat16)
    wo = Wo.astype(jnp.bfloat16)
    kh = (K_ext[0].astype(jnp.bfloat16)
          .reshape(8, NG, 64, HQ, DH)
          .transpose(3, 1, 0, 2, 4)
          .reshape(HQ, NG, GK, DH))
    vh = (V_ext[0].astype(jnp.bfloat16)
          .reshape(8, NG, 64, HQ, DH)
          .transpose(3, 1, 0, 2, 4)
          .reshape(HQ, NG, GK, DH))

    outg = pl.pallas_call(
        _body,
        out_shape=jax.ShapeDtypeStruct((NG, GQ, D), jnp.float32),
        in_specs=[
            pl.BlockSpec(memory_space=pltpu.VMEM),
            pl.BlockSpec(memory_space=pltpu.VMEM),
            pl.BlockSpec(memory_space=pl.ANY),
            pl.BlockSpec(memory_space=pl.ANY),
            pl.BlockSpec(memory_space=pltpu.VMEM),
        ],
        out_specs=pl.BlockSpec(memory_space=pltpu.VMEM),
        scratch_shapes=[
            pltpu.VMEM((N_DEV, 2, 2, D // 2, D), jnp.bfloat16),
            pltpu.VMEM((2, HP, NG, GK, DH), jnp.bfloat16),
            pltpu.VMEM((2, HP, NG, GK, DH), jnp.bfloat16),
            pltpu.SemaphoreType.DMA((4,)),
            pltpu.SemaphoreType.DMA((4,)),
            pltpu.SemaphoreType.DMA((2,)),
            pltpu.SemaphoreType.DMA((2,)),
        ],
        compiler_params=pltpu.CompilerParams(
            collective_id=0,
            vmem_limit_bytes=100 * 1024 * 1024,
        ),
    )(xg, wq, kh, vh, wo)

    out = (outg.reshape(NG, QB, 64, D)
           .transpose(1, 0, 2, 3)
           .reshape(1, SQ, D))
    return out
